# Initial kernel scaffold; baseline (speedup 1.0000x reference)
import jax
import jax.numpy as jnp
from jax.experimental import pallas as pl


def kernel(x, edge_index, W1, b1, W2, b2):
    raise NotImplementedError("write your pallas kernel here")


# speedup vs baseline: 7.0263x; 7.0263x over previous
import functools

import jax
import jax.numpy as jnp
from jax import lax
from jax.experimental import pallas as pl
from jax.experimental.pallas import tpu as pltpu
from jax.experimental.pallas import tpu_sc as plsc


def _mm_body(x_ref, w_ref, o_ref):
    o_ref[...] = jnp.dot(x_ref[...], w_ref[...], preferred_element_type=jnp.float32)


def _matmul(x, w, block_rows=400):
    n, k = x.shape
    m = w.shape[1]
    return pl.pallas_call(
        _mm_body,
        grid=(n // block_rows,),
        in_specs=[
            pl.BlockSpec((block_rows, k), lambda i: (i, 0)),
            pl.BlockSpec((k, m), lambda i: (0, 0)),
        ],
        out_specs=pl.BlockSpec((block_rows, m), lambda i: (i, 0)),
        out_shape=jax.ShapeDtypeStruct((n, m), jnp.float32),
    )(x, w)


def _dinv_from_partials(degp):
    deg = degp[0, :, 0:1] + degp[1, :, 0:1] + 1.0
    return lax.rsqrt(deg)


def _scale_body(h_ref, degp_ref, o_ref):
    o_ref[...] = h_ref[...] * _dinv_from_partials(degp_ref[...])


def _scale(h, degp, block_rows=400):
    n, d = h.shape
    return pl.pallas_call(
        _scale_body,
        grid=(n // block_rows,),
        in_specs=[
            pl.BlockSpec((block_rows, d), lambda i: (i, 0)),
            pl.BlockSpec((2, block_rows, 16), lambda i: (0, i, 0)),
        ],
        out_specs=pl.BlockSpec((block_rows, d), lambda i: (i, 0)),
        out_shape=jax.ShapeDtypeStruct((n, d), jnp.float32),
    )(h, degp)


def _layer2_body(acc_ref, degp_ref, b1_ref, w2_ref, o_ref):
    dinv = _dinv_from_partials(degp_ref[...])
    g = jnp.maximum(acc_ref[...] * dinv + b1_ref[...], 0.0)
    h2 = jnp.dot(g, w2_ref[...], preferred_element_type=jnp.float32)
    o_ref[...] = h2 * dinv


def _layer2(acc, degp, b1, w2, block_rows=400):
    n, d = acc.shape
    m = w2.shape[1]
    return pl.pallas_call(
        _layer2_body,
        grid=(n // block_rows,),
        in_specs=[
            pl.BlockSpec((block_rows, d), lambda i: (i, 0)),
            pl.BlockSpec((2, block_rows, 16), lambda i: (0, i, 0)),
            pl.BlockSpec((1, d), lambda i: (0, 0)),
            pl.BlockSpec((d, m), lambda i: (0, 0)),
        ],
        out_specs=pl.BlockSpec((block_rows, m), lambda i: (i, 0)),
        out_shape=jax.ShapeDtypeStruct((n, m), jnp.float32),
    )(acc, degp, b1, w2)


def _out_body(acc_ref, degp_ref, b2_ref, o_ref):
    dinv = _dinv_from_partials(degp_ref[...])
    o = acc_ref[...] * dinv + b2_ref[...]
    m = jnp.max(o, axis=1, keepdims=True)
    e = jnp.exp(o - m)
    lse = jnp.log(jnp.sum(e, axis=1, keepdims=True)) + m
    o_ref[...] = o - lse


def _out_layer(acc, degp, b2, block_rows=400):
    n, d = acc.shape
    return pl.pallas_call(
        _out_body,
        grid=(n // block_rows,),
        in_specs=[
            pl.BlockSpec((block_rows, d), lambda i: (i, 0)),
            pl.BlockSpec((2, block_rows, 16), lambda i: (0, i, 0)),
            pl.BlockSpec((1, d), lambda i: (0, 0)),
        ],
        out_specs=pl.BlockSpec((block_rows, d), lambda i: (i, 0)),
        out_shape=jax.ShapeDtypeStruct((n, d), jnp.float32),
    )(acc, degp, b2)


@functools.lru_cache(maxsize=None)
def _make_deg_kernel(n_nodes, n_edges):
    info = plsc.get_sparse_core_info()
    nc, ns = info.num_cores, info.num_subcores
    nw = nc * ns
    per_w = n_edges // nw
    chunk = 40
    iters = per_w // chunk
    rch = 80
    n_rch = n_nodes // rch
    k_max = -(-n_rch // ns)
    mesh = plsc.VectorSubcoreMesh(core_axis_name="c", subcore_axis_name="s")

    @functools.partial(
        pl.kernel,
        out_type=jax.ShapeDtypeStruct((nc, n_nodes, 16), jnp.float32),
        mesh=mesh,
        scratch_types=[
            pltpu.VMEM((chunk,), jnp.int32),
            pltpu.VMEM((chunk, 16), jnp.float32),
            pltpu.VMEM((rch, 16), jnp.float32),
            pltpu.VMEM_SHARED((n_nodes, 16), jnp.float32),
        ],
        compiler_params=pltpu.CompilerParams(use_tc_tiling_on_sc=False),
    )
    def deg_kernel(dst_hbm, zeros_hbm, ones_hbm, out_hbm,
                   idx_v, ones_v, bounce_v, acc_sh):
        c = lax.axis_index("c")
        s = lax.axis_index("s")
        wid = s * nc + c
        pltpu.sync_copy(zeros_hbm, bounce_v)
        for k in range(k_max):
            cid = s + k * ns
            if n_rch % ns == 0 or k < k_max - 1:
                pltpu.sync_copy(bounce_v, acc_sh.at[pl.ds(cid * rch, rch)])
            else:
                @pl.when(cid < n_rch)
                def _():
                    pltpu.sync_copy(bounce_v, acc_sh.at[pl.ds(cid * rch, rch)])
        pltpu.sync_copy(ones_hbm, ones_v)
        plsc.subcore_barrier()
        base = wid * per_w

        def body(j, carry):
            pltpu.sync_copy(dst_hbm.at[pl.ds(base + j * chunk, chunk)], idx_v)
            pltpu.sync_copy(ones_v, acc_sh.at[idx_v], add=True)
            return carry

        lax.fori_loop(0, iters, body, 0)
        plsc.subcore_barrier()
        for k in range(k_max):
            cid = s + k * ns

            def _drain(cid=cid):
                pltpu.sync_copy(acc_sh.at[pl.ds(cid * rch, rch)], bounce_v)
                pltpu.sync_copy(bounce_v, out_hbm.at[c, pl.ds(cid * rch, rch)])

            if n_rch % ns == 0 or k < k_max - 1:
                _drain()
            else:
                pl.when(cid < n_rch)(_drain)

    return deg_kernel


@functools.lru_cache(maxsize=None)
def _make_agg_kernel(n_nodes, n_edges, dhalf):
    info = plsc.get_sparse_core_info()
    nc, ns = info.num_cores, info.num_subcores
    per_s = n_edges // ns
    chunk = 80
    iters = per_s // chunk
    rch = 80
    n_rch = n_nodes // rch
    k_max = -(-n_rch // ns)
    mesh = plsc.VectorSubcoreMesh(core_axis_name="c", subcore_axis_name="s")

    @functools.partial(
        pl.kernel,
        out_type=jax.ShapeDtypeStruct((nc, n_nodes, dhalf), jnp.float32),
        mesh=mesh,
        scratch_types=[
            pltpu.VMEM((chunk,), jnp.int32),
            pltpu.VMEM((chunk,), jnp.int32),
            pltpu.VMEM((chunk, dhalf), jnp.float32),
            pltpu.VMEM((rch, dhalf), jnp.float32),
            pltpu.VMEM_SHARED((n_nodes, dhalf), jnp.float32),
            pltpu.SemaphoreType.DMA,
        ],
        compiler_params=pltpu.CompilerParams(use_tc_tiling_on_sc=False),
    )
    def agg_kernel(hsplit_hbm, src_hbm, dst_hbm, out_hbm,
                   src_v, dst_v, rows_v, bounce_v, acc_sh, sem):
        c = lax.axis_index("c")
        s = lax.axis_index("s")

        for k in range(k_max):
            cid = s + k * ns

            def _init(cid=cid):
                pltpu.sync_copy(hsplit_hbm.at[c, pl.ds(cid * rch, rch)], bounce_v)
                pltpu.sync_copy(bounce_v, acc_sh.at[pl.ds(cid * rch, rch)])

            if n_rch % ns == 0 or k < k_max - 1:
                _init()
            else:
                pl.when(cid < n_rch)(_init)
        plsc.subcore_barrier()
        ebase = s * per_s

        def body(j, carry):
            pltpu.sync_copy(src_hbm.at[pl.ds(ebase + j * chunk, chunk)], src_v)
            pltpu.sync_copy(dst_hbm.at[pl.ds(ebase + j * chunk, chunk)], dst_v)
            pltpu.async_copy(hsplit_hbm.at[c].at[src_v], rows_v, sem).wait()
            pltpu.sync_copy(rows_v, acc_sh.at[dst_v], add=True)
            return carry

        lax.fori_loop(0, iters, body, 0)
        plsc.subcore_barrier()

        for k in range(k_max):
            cid = s + k * ns

            def _drain(cid=cid):
                pltpu.sync_copy(acc_sh.at[pl.ds(cid * rch, rch)], bounce_v)
                pltpu.sync_copy(bounce_v, out_hbm.at[c, pl.ds(cid * rch, rch)])

            if n_rch % ns == 0 or k < k_max - 1:
                _drain()
            else:
                pl.when(cid < n_rch)(_drain)

    return agg_kernel


def kernel(x, edge_index, W1, b1, W2, b2):
    n, d_in = x.shape
    d_hid = W1.shape[1]
    d_out = W2.shape[1]
    e = edge_index.shape[1]
    src = edge_index[0].astype(jnp.int32)
    dst = edge_index[1].astype(jnp.int32)

    info = plsc.get_sparse_core_info()
    ns = info.num_subcores
    zeros_aux = jnp.zeros((80, 16), jnp.float32)
    ones_aux = jnp.ones((40, 16), jnp.float32)

    h1 = _matmul(x, W1)
    degp = _make_deg_kernel(n, e)(dst, zeros_aux, ones_aux)
    hp = _scale(h1, degp)

    hsplit = hp.reshape(n, 2, d_hid // 2).transpose(1, 0, 2)
    acc1 = _make_agg_kernel(n, e, d_hid // 2)(hsplit, src, dst)
    acc1 = acc1.transpose(1, 0, 2).reshape(n, d_hid)

    h2p = _layer2(acc1, degp, b1.reshape(1, -1), W2)

    hsplit2 = h2p.reshape(n, 2, d_out // 2).transpose(1, 0, 2)
    acc2 = _make_agg_kernel(n, e, d_out // 2)(hsplit2, src, dst)
    acc2 = acc2.transpose(1, 0, 2).reshape(n, d_out)

    return _out_layer(acc2, degp, b2.reshape(1, -1))


# speedup vs baseline: 9.2196x; 1.3122x over previous
import functools

import jax
import jax.numpy as jnp
from jax import lax
from jax.experimental import pallas as pl
from jax.experimental.pallas import tpu as pltpu
from jax.experimental.pallas import tpu_sc as plsc


def _mm_body(x_ref, w_ref, o_ref):
    o_ref[...] = jnp.dot(x_ref[...], w_ref[...], preferred_element_type=jnp.float32)


def _matmul(x, w, block_rows=400):
    n, k = x.shape
    m = w.shape[1]
    return pl.pallas_call(
        _mm_body,
        grid=(n // block_rows,),
        in_specs=[
            pl.BlockSpec((block_rows, k), lambda i: (i, 0)),
            pl.BlockSpec((k, m), lambda i: (0, 0)),
        ],
        out_specs=pl.BlockSpec((block_rows, m), lambda i: (i, 0)),
        out_shape=jax.ShapeDtypeStruct((n, m), jnp.float32),
    )(x, w)


def _dinv_from_partials(degp):
    deg = degp[0, :, 0:1] + degp[1, :, 0:1] + 1.0
    return lax.rsqrt(deg)


def _scale_body(h_ref, degp_ref, o_ref):
    o_ref[...] = h_ref[...] * _dinv_from_partials(degp_ref[...])


def _scale(h, degp, block_rows=400):
    n, d = h.shape
    return pl.pallas_call(
        _scale_body,
        grid=(n // block_rows,),
        in_specs=[
            pl.BlockSpec((block_rows, d), lambda i: (i, 0)),
            pl.BlockSpec((2, block_rows, 16), lambda i: (0, i, 0)),
        ],
        out_specs=pl.BlockSpec((block_rows, d), lambda i: (i, 0)),
        out_shape=jax.ShapeDtypeStruct((n, d), jnp.float32),
    )(h, degp)


def _layer2_body(acc_ref, degp_ref, b1_ref, w2_ref, o_ref):
    dinv = _dinv_from_partials(degp_ref[...])
    g = jnp.maximum(acc_ref[...] * dinv + b1_ref[...], 0.0)
    h2 = jnp.dot(g, w2_ref[...], preferred_element_type=jnp.float32)
    o_ref[...] = h2 * dinv


def _layer2(acc, degp, b1, w2, block_rows=400):
    n, d = acc.shape
    m = w2.shape[1]
    return pl.pallas_call(
        _layer2_body,
        grid=(n // block_rows,),
        in_specs=[
            pl.BlockSpec((block_rows, d), lambda i: (i, 0)),
            pl.BlockSpec((2, block_rows, 16), lambda i: (0, i, 0)),
            pl.BlockSpec((1, d), lambda i: (0, 0)),
            pl.BlockSpec((d, m), lambda i: (0, 0)),
        ],
        out_specs=pl.BlockSpec((block_rows, m), lambda i: (i, 0)),
        out_shape=jax.ShapeDtypeStruct((n, m), jnp.float32),
    )(acc, degp, b1, w2)


def _out_body(acc_ref, degp_ref, b2_ref, o_ref):
    dinv = _dinv_from_partials(degp_ref[...])
    o = acc_ref[...] * dinv + b2_ref[...]
    m = jnp.max(o, axis=1, keepdims=True)
    e = jnp.exp(o - m)
    lse = jnp.log(jnp.sum(e, axis=1, keepdims=True)) + m
    o_ref[...] = o - lse


def _out_layer(acc, degp, b2, block_rows=400):
    n, d = acc.shape
    return pl.pallas_call(
        _out_body,
        grid=(n // block_rows,),
        in_specs=[
            pl.BlockSpec((block_rows, d), lambda i: (i, 0)),
            pl.BlockSpec((2, block_rows, 16), lambda i: (0, i, 0)),
            pl.BlockSpec((1, d), lambda i: (0, 0)),
        ],
        out_specs=pl.BlockSpec((block_rows, d), lambda i: (i, 0)),
        out_shape=jax.ShapeDtypeStruct((n, d), jnp.float32),
    )(acc, degp, b2)


@functools.lru_cache(maxsize=None)
def _make_deg_kernel(n_nodes, n_edges):
    info = plsc.get_sparse_core_info()
    nc, ns = info.num_cores, info.num_subcores
    nw = nc * ns
    per_w = n_edges // nw
    chunk = 40
    iters = per_w // chunk
    rch = 80
    n_rch = n_nodes // rch
    k_max = -(-n_rch // ns)
    mesh = plsc.VectorSubcoreMesh(core_axis_name="c", subcore_axis_name="s")

    @functools.partial(
        pl.kernel,
        out_type=jax.ShapeDtypeStruct((nc, n_nodes, 16), jnp.float32),
        mesh=mesh,
        scratch_types=[
            pltpu.VMEM((chunk,), jnp.int32),
            pltpu.VMEM((chunk, 16), jnp.float32),
            pltpu.VMEM((rch, 16), jnp.float32),
            pltpu.VMEM_SHARED((n_nodes, 16), jnp.float32),
        ],
        compiler_params=pltpu.CompilerParams(use_tc_tiling_on_sc=False),
    )
    def deg_kernel(dst_hbm, zeros_hbm, ones_hbm, out_hbm,
                   idx_v, ones_v, bounce_v, acc_sh):
        c = lax.axis_index("c")
        s = lax.axis_index("s")
        wid = s * nc + c
        pltpu.sync_copy(zeros_hbm, bounce_v)
        for k in range(k_max):
            cid = s + k * ns
            if n_rch % ns == 0 or k < k_max - 1:
                pltpu.sync_copy(bounce_v, acc_sh.at[pl.ds(cid * rch, rch)])
            else:
                @pl.when(cid < n_rch)
                def _():
                    pltpu.sync_copy(bounce_v, acc_sh.at[pl.ds(cid * rch, rch)])
        pltpu.sync_copy(ones_hbm, ones_v)
        plsc.subcore_barrier()
        base = wid * per_w

        def body(j, carry):
            pltpu.sync_copy(dst_hbm.at[pl.ds(base + j * chunk, chunk)], idx_v)
            pltpu.sync_copy(ones_v, acc_sh.at[idx_v], add=True)
            return carry

        lax.fori_loop(0, iters, body, 0)
        plsc.subcore_barrier()
        for k in range(k_max):
            cid = s + k * ns

            def _drain(cid=cid):
                pltpu.sync_copy(acc_sh.at[pl.ds(cid * rch, rch)], bounce_v)
                pltpu.sync_copy(bounce_v, out_hbm.at[c, pl.ds(cid * rch, rch)])

            if n_rch % ns == 0 or k < k_max - 1:
                _drain()
            else:
                pl.when(cid < n_rch)(_drain)

    return deg_kernel


_CHUNK = 80
_NBUF = 4


@functools.lru_cache(maxsize=None)
def _make_agg_kernel(n_nodes, n_chunks_per_tile, dhalf):
    info = plsc.get_sparse_core_info()
    nc, ns = info.num_cores, info.num_subcores
    n_acc = n_nodes + 8
    rch = _CHUNK
    n_rch = n_nodes // rch
    k_max = -(-n_rch // ns)
    nbuf = _NBUF
    assert n_chunks_per_tile % nbuf == 0
    n_rounds = n_chunks_per_tile // nbuf
    mesh = plsc.VectorSubcoreMesh(core_axis_name="c", subcore_axis_name="s")

    @functools.partial(
        pl.kernel,
        out_type=jax.ShapeDtypeStruct((nc, n_nodes, dhalf), jnp.float32),
        mesh=mesh,
        scratch_types=[
            [pltpu.VMEM((nbuf, _CHUNK), jnp.int32) for _ in range(2)],
            [pltpu.VMEM((nbuf, _CHUNK), jnp.int32) for _ in range(2)],
            [pltpu.VMEM((_CHUNK, dhalf), jnp.float32) for _ in range(nbuf)],
            pltpu.VMEM_SHARED((n_acc, dhalf), jnp.float32),
            [pltpu.SemaphoreType.DMA for _ in range(nbuf)],
            [pltpu.SemaphoreType.DMA for _ in range(nbuf)],
            [pltpu.SemaphoreType.DMA for _ in range(2)],
            [pltpu.SemaphoreType.DMA for _ in range(2)],
        ],
        compiler_params=pltpu.CompilerParams(use_tc_tiling_on_sc=False),
    )
    def agg_kernel(hsplit_hbm, src_hbm, dst_hbm, out_hbm,
                   src_blk, dst_blk, rows_v, acc_sh, gsem, ssem, isem, jsem):
        c = lax.axis_index("c")
        s = lax.axis_index("s")
        table = hsplit_hbm.at[c]
        cbase = s * n_chunks_per_tile
        bounce = rows_v[0]

        for k in range(k_max):
            cid = s + k * ns

            def _init(cid=cid):
                pltpu.sync_copy(hsplit_hbm.at[c, pl.ds(cid * rch, rch)], bounce)
                pltpu.sync_copy(bounce, acc_sh.at[pl.ds(cid * rch, rch)])

            if n_rch % ns == 0 or k < k_max - 1:
                _init()
            else:
                pl.when(cid < n_rch)(_init)
        plsc.subcore_barrier()

        pltpu.sync_copy(src_hbm.at[pl.ds(cbase, nbuf)], src_blk[0])
        pltpu.sync_copy(dst_hbm.at[pl.ds(cbase, nbuf)], dst_blk[0])
        for b in range(nbuf):
            pltpu.async_copy(table.at[src_blk[0].at[b]], rows_v[b], gsem[b])

        def round_body(g, carry):
            i = lax.rem(g, 2)

            @pl.when(g + 1 < n_rounds)
            def _():
                nxt = cbase + (g + 1) * nbuf
                for t in range(2):
                    @pl.when(i == 1 - t)
                    def _(t=t):
                        pltpu.async_copy(src_hbm.at[pl.ds(nxt, nbuf)],
                                         src_blk[t], isem[t])
                        pltpu.async_copy(dst_hbm.at[pl.ds(nxt, nbuf)],
                                         dst_blk[t], jsem[t])

            for t in range(2):
                @pl.when(i == t)
                def _(t=t):
                    for b in range(nbuf):
                        pltpu.make_async_copy(table.at[src_blk[t].at[b]],
                                              rows_v[b], gsem[b]).wait()
                        pltpu.async_copy(rows_v[b],
                                         acc_sh.at[dst_blk[t].at[b]],
                                         ssem[b], add=True)
                    for b in range(nbuf):
                        pltpu.make_async_copy(rows_v[b],
                                              acc_sh.at[dst_blk[t].at[b]],
                                              ssem[b]).wait()

                        @pl.when(g + 1 < n_rounds)
                        def _(b=b, t=t):
                            if b == 0:
                                pltpu.make_async_copy(
                                    src_hbm.at[pl.ds(cbase, nbuf)],
                                    src_blk[1 - t], isem[1 - t]).wait()
                                pltpu.make_async_copy(
                                    dst_hbm.at[pl.ds(cbase, nbuf)],
                                    dst_blk[1 - t], jsem[1 - t]).wait()
                            pltpu.async_copy(table.at[src_blk[1 - t].at[b]],
                                             rows_v[b], gsem[b])
            return carry

        lax.fori_loop(0, n_rounds, round_body, 0)
        plsc.subcore_barrier()

        for k in range(k_max):
            cid = s + k * ns

            def _drain(cid=cid):
                pltpu.sync_copy(acc_sh.at[pl.ds(cid * rch, rch)], bounce)
                pltpu.sync_copy(bounce, out_hbm.at[c, pl.ds(cid * rch, rch)])

            if n_rch % ns == 0 or k < k_max - 1:
                _drain()
            else:
                pl.when(cid < n_rch)(_drain)

    return agg_kernel


def kernel(x, edge_index, W1, b1, W2, b2):
    n, d_in = x.shape
    d_hid = W1.shape[1]
    d_out = W2.shape[1]
    e = edge_index.shape[1]
    src = edge_index[0].astype(jnp.int32)
    dst = edge_index[1].astype(jnp.int32)

    info = plsc.get_sparse_core_info()
    ns = info.num_subcores
    zeros_aux = jnp.zeros((80, 16), jnp.float32)
    ones_aux = jnp.ones((40, 16), jnp.float32)

    per_tile = e // ns
    n_chunks = -(-per_tile // _CHUNK)
    n_chunks += (-n_chunks) % _NBUF
    pad = n_chunks * _CHUNK - per_tile
    fill = jnp.full((ns, pad), n, dtype=jnp.int32)
    src2 = jnp.concatenate([src.reshape(ns, per_tile), fill], axis=1)
    src2 = src2.reshape(ns * n_chunks, _CHUNK)
    dst2 = jnp.concatenate([dst.reshape(ns, per_tile), fill], axis=1)
    dst2 = dst2.reshape(ns * n_chunks, _CHUNK)

    h1 = _matmul(x, W1)
    degp = _make_deg_kernel(n, e)(dst, zeros_aux, ones_aux)
    hp = _scale(h1, degp)

    def agg(hmat, d):
        hsplit = hmat.reshape(n, 2, d // 2).transpose(1, 0, 2)
        hsplit = jnp.concatenate(
            [hsplit, jnp.zeros((2, 8, d // 2), jnp.float32)], axis=1)
        acc = _make_agg_kernel(n, n_chunks, d // 2)(hsplit, src2, dst2)
        return acc.transpose(1, 0, 2).reshape(n, d)

    acc1 = agg(hp, d_hid)
    h2p = _layer2(acc1, degp, b1.reshape(1, -1), W2)
    acc2 = agg(h2p, d_out)
    return _out_layer(acc2, degp, b2.reshape(1, -1))


# speedup vs baseline: 9.9318x; 1.0773x over previous
import functools

import jax
import jax.numpy as jnp
from jax import lax
from jax.experimental import pallas as pl
from jax.experimental.pallas import tpu as pltpu
from jax.experimental.pallas import tpu_sc as plsc


def _mm_body(x_ref, w_ref, o_ref):
    o_ref[...] = jnp.dot(x_ref[...], w_ref[...], preferred_element_type=jnp.float32)


def _matmul(x, w, block_rows=400):
    n, k = x.shape
    m = w.shape[1]
    return pl.pallas_call(
        _mm_body,
        grid=(n // block_rows,),
        in_specs=[
            pl.BlockSpec((block_rows, k), lambda i: (i, 0)),
            pl.BlockSpec((k, m), lambda i: (0, 0)),
        ],
        out_specs=pl.BlockSpec((block_rows, m), lambda i: (i, 0)),
        out_shape=jax.ShapeDtypeStruct((n, m), jnp.float32),
    )(x, w)


def _dinv_from_partials(degp):
    deg = degp[0, :, 0:1] + degp[1, :, 0:1] + 1.0
    return lax.rsqrt(deg)


def _scale_body(h_ref, degp_ref, o_ref):
    o_ref[...] = h_ref[...] * _dinv_from_partials(degp_ref[...])


def _scale(h, degp, block_rows=400):
    n, d = h.shape
    return pl.pallas_call(
        _scale_body,
        grid=(n // block_rows,),
        in_specs=[
            pl.BlockSpec((block_rows, d), lambda i: (i, 0)),
            pl.BlockSpec((2, block_rows, 16), lambda i: (0, i, 0)),
        ],
        out_specs=pl.BlockSpec((block_rows, d), lambda i: (i, 0)),
        out_shape=jax.ShapeDtypeStruct((n, d), jnp.float32),
    )(h, degp)


def _layer2_body(acc_ref, degp_ref, b1_ref, w2_ref, o_ref):
    dinv = _dinv_from_partials(degp_ref[...])
    g = jnp.maximum(acc_ref[...] * dinv + b1_ref[...], 0.0)
    h2 = jnp.dot(g, w2_ref[...], preferred_element_type=jnp.float32)
    o_ref[...] = h2 * dinv


def _layer2(acc, degp, b1, w2, block_rows=400):
    n, d = acc.shape
    m = w2.shape[1]
    return pl.pallas_call(
        _layer2_body,
        grid=(n // block_rows,),
        in_specs=[
            pl.BlockSpec((block_rows, d), lambda i: (i, 0)),
            pl.BlockSpec((2, block_rows, 16), lambda i: (0, i, 0)),
            pl.BlockSpec((1, d), lambda i: (0, 0)),
            pl.BlockSpec((d, m), lambda i: (0, 0)),
        ],
        out_specs=pl.BlockSpec((block_rows, m), lambda i: (i, 0)),
        out_shape=jax.ShapeDtypeStruct((n, m), jnp.float32),
    )(acc, degp, b1, w2)


def _out_body(acc_ref, degp_ref, b2_ref, o_ref):
    dinv = _dinv_from_partials(degp_ref[...])
    o = acc_ref[...] * dinv + b2_ref[...]
    m = jnp.max(o, axis=1, keepdims=True)
    e = jnp.exp(o - m)
    lse = jnp.log(jnp.sum(e, axis=1, keepdims=True)) + m
    o_ref[...] = o - lse


def _out_layer(acc, degp, b2, block_rows=400):
    n, d = acc.shape
    return pl.pallas_call(
        _out_body,
        grid=(n // block_rows,),
        in_specs=[
            pl.BlockSpec((block_rows, d), lambda i: (i, 0)),
            pl.BlockSpec((2, block_rows, 16), lambda i: (0, i, 0)),
            pl.BlockSpec((1, d), lambda i: (0, 0)),
        ],
        out_specs=pl.BlockSpec((block_rows, d), lambda i: (i, 0)),
        out_shape=jax.ShapeDtypeStruct((n, d), jnp.float32),
    )(acc, degp, b2)


_DCHUNK = 128


@functools.lru_cache(maxsize=None)
def _make_deg_kernel(n_nodes, n_chunks_per_w):
    info = plsc.get_sparse_core_info()
    nc, ns = info.num_cores, info.num_subcores
    n_acc = n_nodes + 8
    rch = 80
    n_rch = n_nodes // rch
    k_max = -(-n_rch // ns)
    nbuf = _NBUF
    assert n_chunks_per_w % nbuf == 0
    n_rounds = n_chunks_per_w // nbuf
    mesh = plsc.VectorSubcoreMesh(core_axis_name="c", subcore_axis_name="s")

    @functools.partial(
        pl.kernel,
        out_type=jax.ShapeDtypeStruct((nc, n_nodes, 16), jnp.float32),
        mesh=mesh,
        scratch_types=[
            [pltpu.VMEM((nbuf, _DCHUNK), jnp.int32) for _ in range(2)],
            pltpu.VMEM((_DCHUNK, 16), jnp.float32),
            pltpu.VMEM_SHARED((n_acc, 16), jnp.float32),
            pltpu.SemaphoreType.DMA,
            [pltpu.SemaphoreType.DMA for _ in range(2)],
        ],
        compiler_params=pltpu.CompilerParams(use_tc_tiling_on_sc=False),
    )
    def deg_kernel(dst_hbm, zeros_hbm, ones_hbm, out_hbm,
                   dst_blk, ones_v, acc_sh, ssem, isem):
        c = lax.axis_index("c")
        s = lax.axis_index("s")
        wid = s * nc + c
        cbase = wid * n_chunks_per_w
        for k in range(k_max):
            cid = s + k * ns
            if n_rch % ns == 0 or k < k_max - 1:
                pltpu.sync_copy(zeros_hbm, acc_sh.at[pl.ds(cid * rch, rch)])
            else:
                @pl.when(cid < n_rch)
                def _():
                    pltpu.sync_copy(zeros_hbm, acc_sh.at[pl.ds(cid * rch, rch)])
        pltpu.sync_copy(ones_hbm, ones_v)
        plsc.subcore_barrier()

        pltpu.sync_copy(dst_hbm.at[pl.ds(cbase, nbuf)], dst_blk[0])

        def round_body(g, carry):
            i = lax.rem(g, 2)

            @pl.when(g + 1 < n_rounds)
            def _():
                nxt = cbase + (g + 1) * nbuf
                for t in range(2):
                    @pl.when(i == 1 - t)
                    def _(t=t):
                        pltpu.async_copy(dst_hbm.at[pl.ds(nxt, nbuf)],
                                         dst_blk[t], isem[t])

            for t in range(2):
                @pl.when(i == t)
                def _(t=t):
                    for b in range(nbuf):
                        pltpu.async_copy(ones_v, acc_sh.at[dst_blk[t].at[b]],
                                         ssem, add=True)
                    for b in range(nbuf):
                        pltpu.make_async_copy(ones_v,
                                              acc_sh.at[dst_blk[t].at[b]],
                                              ssem).wait()

                    @pl.when(g + 1 < n_rounds)
                    def _(t=t):
                        pltpu.make_async_copy(dst_hbm.at[pl.ds(cbase, nbuf)],
                                              dst_blk[1 - t], isem[1 - t]).wait()
            return carry

        lax.fori_loop(0, n_rounds, round_body, 0)
        plsc.subcore_barrier()
        for k in range(k_max):
            cid = s + k * ns

            def _drain(cid=cid):
                pltpu.sync_copy(acc_sh.at[pl.ds(cid * rch, rch)],
                                out_hbm.at[c, pl.ds(cid * rch, rch)])

            if n_rch % ns == 0 or k < k_max - 1:
                _drain()
            else:
                pl.when(cid < n_rch)(_drain)

    return deg_kernel


_CHUNK = 80
_NBUF = 4


@functools.lru_cache(maxsize=None)
def _make_agg_kernel(n_nodes, n_chunks_per_tile, dhalf):
    info = plsc.get_sparse_core_info()
    nc, ns = info.num_cores, info.num_subcores
    n_acc = n_nodes + 8
    rch = _CHUNK
    n_rch = n_nodes // rch
    k_max = -(-n_rch // ns)
    nbuf = _NBUF
    assert n_chunks_per_tile % nbuf == 0
    n_rounds = n_chunks_per_tile // nbuf
    mesh = plsc.VectorSubcoreMesh(core_axis_name="c", subcore_axis_name="s")

    @functools.partial(
        pl.kernel,
        out_type=jax.ShapeDtypeStruct((nc, n_nodes, dhalf), jnp.float32),
        mesh=mesh,
        scratch_types=[
            [pltpu.VMEM((nbuf, _CHUNK), jnp.int32) for _ in range(2)],
            [pltpu.VMEM((nbuf, _CHUNK), jnp.int32) for _ in range(2)],
            [pltpu.VMEM((_CHUNK, dhalf), jnp.float32) for _ in range(nbuf)],
            pltpu.VMEM_SHARED((n_acc, dhalf), jnp.float32),
            [pltpu.SemaphoreType.DMA for _ in range(nbuf)],
            [pltpu.SemaphoreType.DMA for _ in range(nbuf)],
            [pltpu.SemaphoreType.DMA for _ in range(2)],
            [pltpu.SemaphoreType.DMA for _ in range(2)],
        ],
        compiler_params=pltpu.CompilerParams(use_tc_tiling_on_sc=False),
    )
    def agg_kernel(hsplit_hbm, src_hbm, dst_hbm, out_hbm,
                   src_blk, dst_blk, rows_v, acc_sh, gsem, ssem, isem, jsem):
        c = lax.axis_index("c")
        s = lax.axis_index("s")
        table = hsplit_hbm.at[c]
        cbase = s * n_chunks_per_tile
        bounce = rows_v[0]

        for k in range(k_max):
            cid = s + k * ns

            def _init(cid=cid):
                pltpu.sync_copy(hsplit_hbm.at[c, pl.ds(cid * rch, rch)],
                                acc_sh.at[pl.ds(cid * rch, rch)])

            if n_rch % ns == 0 or k < k_max - 1:
                _init()
            else:
                pl.when(cid < n_rch)(_init)
        plsc.subcore_barrier()

        pltpu.sync_copy(src_hbm.at[pl.ds(cbase, nbuf)], src_blk[0])
        pltpu.sync_copy(dst_hbm.at[pl.ds(cbase, nbuf)], dst_blk[0])
        for b in range(nbuf):
            pltpu.async_copy(table.at[src_blk[0].at[b]], rows_v[b], gsem[b])

        def round_body(g, carry):
            i = lax.rem(g, 2)

            @pl.when(g + 1 < n_rounds)
            def _():
                nxt = cbase + (g + 1) * nbuf
                for t in range(2):
                    @pl.when(i == 1 - t)
                    def _(t=t):
                        pltpu.async_copy(src_hbm.at[pl.ds(nxt, nbuf)],
                                         src_blk[t], isem[t])
                        pltpu.async_copy(dst_hbm.at[pl.ds(nxt, nbuf)],
                                         dst_blk[t], jsem[t])

            for t in range(2):
                @pl.when(i == t)
                def _(t=t):
                    for b in range(nbuf):
                        pltpu.make_async_copy(table.at[src_blk[t].at[b]],
                                              rows_v[b], gsem[b]).wait()
                        pltpu.async_copy(rows_v[b],
                                         acc_sh.at[dst_blk[t].at[b]],
                                         ssem[b], add=True)
                    for b in range(nbuf):
                        pltpu.make_async_copy(rows_v[b],
                                              acc_sh.at[dst_blk[t].at[b]],
                                              ssem[b]).wait()

                        @pl.when(g + 1 < n_rounds)
                        def _(b=b, t=t):
                            if b == 0:
                                pltpu.make_async_copy(
                                    src_hbm.at[pl.ds(cbase, nbuf)],
                                    src_blk[1 - t], isem[1 - t]).wait()
                                pltpu.make_async_copy(
                                    dst_hbm.at[pl.ds(cbase, nbuf)],
                                    dst_blk[1 - t], jsem[1 - t]).wait()
                            pltpu.async_copy(table.at[src_blk[1 - t].at[b]],
                                             rows_v[b], gsem[b])
            return carry

        lax.fori_loop(0, n_rounds, round_body, 0)
        plsc.subcore_barrier()

        for k in range(k_max):
            cid = s + k * ns

            def _drain(cid=cid):
                pltpu.sync_copy(acc_sh.at[pl.ds(cid * rch, rch)],
                                out_hbm.at[c, pl.ds(cid * rch, rch)])

            if n_rch % ns == 0 or k < k_max - 1:
                _drain()
            else:
                pl.when(cid < n_rch)(_drain)

    return agg_kernel


def kernel(x, edge_index, W1, b1, W2, b2):
    n, d_in = x.shape
    d_hid = W1.shape[1]
    d_out = W2.shape[1]
    e = edge_index.shape[1]
    src = edge_index[0].astype(jnp.int32)
    dst = edge_index[1].astype(jnp.int32)

    info = plsc.get_sparse_core_info()
    ns = info.num_subcores
    nw = 2 * ns
    zeros_aux = jnp.zeros((80, 16), jnp.float32)
    ones_aux = jnp.ones((_DCHUNK, 16), jnp.float32)

    per_tile = e // ns
    n_chunks = -(-per_tile // _CHUNK)
    n_chunks += (-n_chunks) % _NBUF
    pad = n_chunks * _CHUNK - per_tile
    fill = jnp.full((ns, pad), n, dtype=jnp.int32)
    src2 = jnp.concatenate([src.reshape(ns, per_tile), fill], axis=1)
    src2 = src2.reshape(ns * n_chunks, _CHUNK)
    dst2 = jnp.concatenate([dst.reshape(ns, per_tile), fill], axis=1)
    dst2 = dst2.reshape(ns * n_chunks, _CHUNK)

    per_w = e // nw
    n_chunks_w = -(-per_w // _DCHUNK)
    n_chunks_w += (-n_chunks_w) % _NBUF
    fill_w = jnp.full((nw, n_chunks_w * _DCHUNK - per_w), n, dtype=jnp.int32)
    dst3 = jnp.concatenate([dst.reshape(nw, per_w), fill_w], axis=1)
    dst3 = dst3.reshape(nw * n_chunks_w, _DCHUNK)

    h1 = _matmul(x, W1)
    degp = _make_deg_kernel(n, n_chunks_w)(dst3, zeros_aux, ones_aux)
    hp = _scale(h1, degp)

    def agg(hmat, d):
        hsplit = hmat.reshape(n, 2, d // 2).transpose(1, 0, 2)
        hsplit = jnp.concatenate(
            [hsplit, jnp.zeros((2, 8, d // 2), jnp.float32)], axis=1)
        acc = _make_agg_kernel(n, n_chunks, d // 2)(hsplit, src2, dst2)
        return acc.transpose(1, 0, 2).reshape(n, d)

    acc1 = agg(hp, d_hid)
    h2p = _layer2(acc1, degp, b1.reshape(1, -1), W2)
    acc2 = agg(h2p, d_out)
    return _out_layer(acc2, degp, b2.reshape(1, -1))


# speedup vs baseline: 11.3289x; 1.1407x over previous
import functools

import jax
import jax.numpy as jnp
from jax import lax
from jax.experimental import pallas as pl
from jax.experimental.pallas import tpu as pltpu
from jax.experimental.pallas import tpu_sc as plsc


def _dinv_from_partials(degp):
    deg = degp[0, :, 0:1] + degp[1, :, 0:1] + 1.0
    return lax.rsqrt(deg)


def _mm_scale_body(x_ref, w_ref, degp_ref, o_ref):
    h = jnp.dot(x_ref[...], w_ref[...], preferred_element_type=jnp.float32)
    h = h * _dinv_from_partials(degp_ref[...])
    d = h.shape[1] // 2
    o_ref[0] = h[:, :d]
    o_ref[1] = h[:, d:]


def _mm_scale(x, w, degp, block_rows=400):
    n, k = x.shape
    m = w.shape[1]
    return pl.pallas_call(
        _mm_scale_body,
        grid=(n // block_rows,),
        in_specs=[
            pl.BlockSpec((block_rows, k), lambda i: (i, 0)),
            pl.BlockSpec((k, m), lambda i: (0, 0)),
            pl.BlockSpec((2, block_rows, 16), lambda i: (0, i, 0)),
        ],
        out_specs=pl.BlockSpec((2, block_rows, m // 2), lambda i: (0, i, 0)),
        out_shape=jax.ShapeDtypeStruct((2, n, m // 2), jnp.float32),
    )(x, w, degp)


def _layer2_body(acc_ref, degp_ref, b1_ref, w2_ref, o_ref):
    dinv = _dinv_from_partials(degp_ref[...])
    acc = jnp.concatenate([acc_ref[0], acc_ref[1]], axis=1)
    g = jnp.maximum(acc * dinv + b1_ref[...], 0.0)
    h2 = jnp.dot(g, w2_ref[...], preferred_element_type=jnp.float32)
    h2 = h2 * dinv
    d = h2.shape[1] // 2
    o_ref[0] = h2[:, :d]
    o_ref[1] = h2[:, d:]


def _layer2(acc, degp, b1, w2, block_rows=400):
    _, n, dh = acc.shape
    d = 2 * dh
    m = w2.shape[1]
    return pl.pallas_call(
        _layer2_body,
        grid=(n // block_rows,),
        in_specs=[
            pl.BlockSpec((2, block_rows, dh), lambda i: (0, i, 0)),
            pl.BlockSpec((2, block_rows, 16), lambda i: (0, i, 0)),
            pl.BlockSpec((1, d), lambda i: (0, 0)),
            pl.BlockSpec((d, m), lambda i: (0, 0)),
        ],
        out_specs=pl.BlockSpec((2, block_rows, m // 2), lambda i: (0, i, 0)),
        out_shape=jax.ShapeDtypeStruct((2, n, m // 2), jnp.float32),
    )(acc, degp, b1, w2)


def _out_body(acc_ref, degp_ref, b2_ref, o_ref):
    dinv = _dinv_from_partials(degp_ref[...])
    acc = jnp.concatenate([acc_ref[0], acc_ref[1]], axis=1)
    o = acc * dinv + b2_ref[...]
    m = jnp.max(o, axis=1, keepdims=True)
    e = jnp.exp(o - m)
    lse = jnp.log(jnp.sum(e, axis=1, keepdims=True)) + m
    o_ref[...] = o - lse


def _out_layer(acc, degp, b2, block_rows=400):
    _, n, dh = acc.shape
    d = 2 * dh
    return pl.pallas_call(
        _out_body,
        grid=(n // block_rows,),
        in_specs=[
            pl.BlockSpec((2, block_rows, dh), lambda i: (0, i, 0)),
            pl.BlockSpec((2, block_rows, 16), lambda i: (0, i, 0)),
            pl.BlockSpec((1, d), lambda i: (0, 0)),
        ],
        out_specs=pl.BlockSpec((block_rows, d), lambda i: (i, 0)),
        out_shape=jax.ShapeDtypeStruct((n, d), jnp.float32),
    )(acc, degp, b2)


_DCHUNK = 128


@functools.lru_cache(maxsize=None)
def _make_deg_kernel(n_nodes, n_chunks_per_w):
    info = plsc.get_sparse_core_info()
    nc, ns = info.num_cores, info.num_subcores
    n_acc = n_nodes + 8
    rch = 80
    n_rch = n_nodes // rch
    k_max = -(-n_rch // ns)
    nbuf = _NBUF
    assert n_chunks_per_w % nbuf == 0
    n_rounds = n_chunks_per_w // nbuf
    mesh = plsc.VectorSubcoreMesh(core_axis_name="c", subcore_axis_name="s")

    @functools.partial(
        pl.kernel,
        out_type=jax.ShapeDtypeStruct((nc, n_nodes, 16), jnp.float32),
        mesh=mesh,
        scratch_types=[
            [pltpu.VMEM((nbuf, _DCHUNK), jnp.int32) for _ in range(2)],
            pltpu.VMEM((_DCHUNK, 16), jnp.float32),
            pltpu.VMEM_SHARED((n_acc, 16), jnp.float32),
            pltpu.SemaphoreType.DMA,
            [pltpu.SemaphoreType.DMA for _ in range(2)],
        ],
        compiler_params=pltpu.CompilerParams(use_tc_tiling_on_sc=False),
    )
    def deg_kernel(dst_hbm, zeros_hbm, ones_hbm, out_hbm,
                   dst_blk, ones_v, acc_sh, ssem, isem):
        c = lax.axis_index("c")
        s = lax.axis_index("s")
        wid = s * nc + c
        cbase = wid * n_chunks_per_w
        for k in range(k_max):
            cid = s + k * ns
            if n_rch % ns == 0 or k < k_max - 1:
                pltpu.sync_copy(zeros_hbm, acc_sh.at[pl.ds(cid * rch, rch)])
            else:
                @pl.when(cid < n_rch)
                def _():
                    pltpu.sync_copy(zeros_hbm, acc_sh.at[pl.ds(cid * rch, rch)])
        pltpu.sync_copy(ones_hbm, ones_v)
        plsc.subcore_barrier()

        pltpu.sync_copy(dst_hbm.at[pl.ds(cbase, nbuf)], dst_blk[0])

        def round_body(g, carry):
            i = lax.rem(g, 2)

            @pl.when(g + 1 < n_rounds)
            def _():
                nxt = cbase + (g + 1) * nbuf
                for t in range(2):
                    @pl.when(i == 1 - t)
                    def _(t=t):
                        pltpu.async_copy(dst_hbm.at[pl.ds(nxt, nbuf)],
                                         dst_blk[t], isem[t])

            for t in range(2):
                @pl.when(i == t)
                def _(t=t):
                    for b in range(nbuf):
                        pltpu.async_copy(ones_v, acc_sh.at[dst_blk[t].at[b]],
                                         ssem, add=True)
                    for b in range(nbuf):
                        pltpu.make_async_copy(ones_v,
                                              acc_sh.at[dst_blk[t].at[b]],
                                              ssem).wait()

                    @pl.when(g + 1 < n_rounds)
                    def _(t=t):
                        pltpu.make_async_copy(dst_hbm.at[pl.ds(cbase, nbuf)],
                                              dst_blk[1 - t], isem[1 - t]).wait()
            return carry

        lax.fori_loop(0, n_rounds, round_body, 0)
        plsc.subcore_barrier()
        for k in range(k_max):
            cid = s + k * ns

            def _drain(cid=cid):
                pltpu.sync_copy(acc_sh.at[pl.ds(cid * rch, rch)],
                                out_hbm.at[c, pl.ds(cid * rch, rch)])

            if n_rch % ns == 0 or k < k_max - 1:
                _drain()
            else:
                pl.when(cid < n_rch)(_drain)

    return deg_kernel


_CHUNK = 80
_NBUF = 4


@functools.lru_cache(maxsize=None)
def _make_agg_kernel(n_nodes, n_chunks_per_tile, dhalf):
    info = plsc.get_sparse_core_info()
    nc, ns = info.num_cores, info.num_subcores
    n_acc = n_nodes + 8
    rch = _CHUNK
    n_rch = n_nodes // rch
    k_max = -(-n_rch // ns)
    nbuf = _NBUF
    assert n_chunks_per_tile % nbuf == 0
    n_rounds = n_chunks_per_tile // nbuf
    mesh = plsc.VectorSubcoreMesh(core_axis_name="c", subcore_axis_name="s")

    @functools.partial(
        pl.kernel,
        out_type=jax.ShapeDtypeStruct((nc, n_nodes, dhalf), jnp.float32),
        mesh=mesh,
        scratch_types=[
            [pltpu.VMEM((nbuf, _CHUNK), jnp.int32) for _ in range(2)],
            [pltpu.VMEM((nbuf, _CHUNK), jnp.int32) for _ in range(2)],
            [pltpu.VMEM((_CHUNK, dhalf), jnp.float32) for _ in range(nbuf)],
            pltpu.VMEM_SHARED((n_acc, dhalf), jnp.float32),
            [pltpu.SemaphoreType.DMA for _ in range(nbuf)],
            [pltpu.SemaphoreType.DMA for _ in range(nbuf)],
            [pltpu.SemaphoreType.DMA for _ in range(2)],
            [pltpu.SemaphoreType.DMA for _ in range(2)],
        ],
        compiler_params=pltpu.CompilerParams(use_tc_tiling_on_sc=False),
    )
    def agg_kernel(hsplit_hbm, src_hbm, dst_hbm, out_hbm,
                   src_blk, dst_blk, rows_v, acc_sh, gsem, ssem, isem, jsem):
        c = lax.axis_index("c")
        s = lax.axis_index("s")
        table = hsplit_hbm.at[c]
        cbase = s * n_chunks_per_tile
        bounce = rows_v[0]

        for k in range(k_max):
            cid = s + k * ns

            def _init(cid=cid):
                pltpu.sync_copy(hsplit_hbm.at[c, pl.ds(cid * rch, rch)],
                                acc_sh.at[pl.ds(cid * rch, rch)])

            if n_rch % ns == 0 or k < k_max - 1:
                _init()
            else:
                pl.when(cid < n_rch)(_init)
        plsc.subcore_barrier()

        pltpu.sync_copy(src_hbm.at[pl.ds(cbase, nbuf)], src_blk[0])
        pltpu.sync_copy(dst_hbm.at[pl.ds(cbase, nbuf)], dst_blk[0])
        for b in range(nbuf):
            pltpu.async_copy(table.at[src_blk[0].at[b]], rows_v[b], gsem[b])

        def round_body(g, carry):
            i = lax.rem(g, 2)

            @pl.when(g + 1 < n_rounds)
            def _():
                nxt = cbase + (g + 1) * nbuf
                for t in range(2):
                    @pl.when(i == 1 - t)
                    def _(t=t):
                        pltpu.async_copy(src_hbm.at[pl.ds(nxt, nbuf)],
                                         src_blk[t], isem[t])
                        pltpu.async_copy(dst_hbm.at[pl.ds(nxt, nbuf)],
                                         dst_blk[t], jsem[t])

            for t in range(2):
                @pl.when(i == t)
                def _(t=t):
                    for b in range(nbuf):
                        pltpu.make_async_copy(table.at[src_blk[t].at[b]],
                                              rows_v[b], gsem[b]).wait()
                        pltpu.async_copy(rows_v[b],
                                         acc_sh.at[dst_blk[t].at[b]],
                                         ssem[b], add=True)
                    for b in range(nbuf):
                        pltpu.make_async_copy(rows_v[b],
                                              acc_sh.at[dst_blk[t].at[b]],
                                              ssem[b]).wait()

                        @pl.when(g + 1 < n_rounds)
                        def _(b=b, t=t):
                            if b == 0:
                                pltpu.make_async_copy(
                                    src_hbm.at[pl.ds(cbase, nbuf)],
                                    src_blk[1 - t], isem[1 - t]).wait()
                                pltpu.make_async_copy(
                                    dst_hbm.at[pl.ds(cbase, nbuf)],
                                    dst_blk[1 - t], jsem[1 - t]).wait()
                            pltpu.async_copy(table.at[src_blk[1 - t].at[b]],
                                             rows_v[b], gsem[b])
            return carry

        lax.fori_loop(0, n_rounds, round_body, 0)
        plsc.subcore_barrier()

        for k in range(k_max):
            cid = s + k * ns

            def _drain(cid=cid):
                pltpu.sync_copy(acc_sh.at[pl.ds(cid * rch, rch)],
                                out_hbm.at[c, pl.ds(cid * rch, rch)])

            if n_rch % ns == 0 or k < k_max - 1:
                _drain()
            else:
                pl.when(cid < n_rch)(_drain)

    return agg_kernel


def kernel(x, edge_index, W1, b1, W2, b2):
    n, d_in = x.shape
    d_hid = W1.shape[1]
    d_out = W2.shape[1]
    e = edge_index.shape[1]
    src = edge_index[0].astype(jnp.int32)
    dst = edge_index[1].astype(jnp.int32)

    info = plsc.get_sparse_core_info()
    ns = info.num_subcores
    nw = 2 * ns
    zeros_aux = jnp.zeros((80, 16), jnp.float32)
    ones_aux = jnp.ones((_DCHUNK, 16), jnp.float32)

    per_tile = e // ns
    n_chunks = -(-per_tile // _CHUNK)
    n_chunks += (-n_chunks) % _NBUF
    pad = n_chunks * _CHUNK - per_tile
    src2 = jnp.concatenate(
        [src.reshape(ns, per_tile), jnp.zeros((ns, pad), jnp.int32)], axis=1)
    src2 = src2.reshape(ns * n_chunks, _CHUNK)
    dst2 = jnp.concatenate(
        [dst.reshape(ns, per_tile), jnp.full((ns, pad), n, jnp.int32)], axis=1)
    dst2 = dst2.reshape(ns * n_chunks, _CHUNK)

    per_w = e // nw
    n_chunks_w = -(-per_w // _DCHUNK)
    n_chunks_w += (-n_chunks_w) % _NBUF
    fill_w = jnp.full((nw, n_chunks_w * _DCHUNK - per_w), n, dtype=jnp.int32)
    dst3 = jnp.concatenate([dst.reshape(nw, per_w), fill_w], axis=1)
    dst3 = dst3.reshape(nw * n_chunks_w, _DCHUNK)

    degp = _make_deg_kernel(n, n_chunks_w)(dst3, zeros_aux, ones_aux)
    hsplit = _mm_scale(x, W1, degp)
    acc1 = _make_agg_kernel(n, n_chunks, d_hid // 2)(hsplit, src2, dst2)
    h2split = _layer2(acc1, degp, b1.reshape(1, -1), W2)
    acc2 = _make_agg_kernel(n, n_chunks, d_out // 2)(h2split, src2, dst2)
    return _out_layer(acc2, degp, b2.reshape(1, -1))
